# Initial kernel scaffold; baseline (speedup 1.0000x reference)
import jax
import jax.numpy as jnp
from jax.experimental import pallas as pl


def kernel(node_features, edge_index, edge_weight, batch, W1, b1, W2, b2):
    raise NotImplementedError("write your pallas kernel here")


# speedup vs baseline: 5.7711x; 5.7711x over previous
import dataclasses
import functools

import jax
import jax.numpy as jnp
from jax import lax
from jax.experimental import pallas as pl
from jax.experimental.pallas import tpu as pltpu
from jax.experimental.pallas import tpu_sc as plsc

N = 10000
E = 320000
DIN = 128
DOUT = 256
DH = DOUT // 2
G = 128

NC = 2
NS = 16
L = 16
K = 128
SB = 16

EPAD = 327680
CH_DEG = EPAD // (NC * NS) // K
CH_AGG = EPAD // NS // K
NB_AGG = CH_AGG // SB

NPAD = 10240
SLICE_N = NPAD // NS
ZR = 64


def _sc_mesh():
    return plsc.VectorSubcoreMesh(core_axis_name="c", subcore_axis_name="s")


def _sc_params():
    cp = pltpu.CompilerParams()
    if "needs_layout_passes" in pltpu.CompilerParams.__dataclass_fields__:
        cp = dataclasses.replace(cp, needs_layout_passes=False)
    return cp


def _deg_sc(col_d, w_d):
    @functools.partial(
        pl.kernel,
        out_type=jax.ShapeDtypeStruct((NC * NPAD,), jnp.float32),
        mesh=_sc_mesh(),
        compiler_params=_sc_params(),
        scratch_types=[
            pltpu.VMEM((SB, K), jnp.int32),
            pltpu.VMEM((SB, K), jnp.float32),
            pltpu.VMEM((SLICE_N,), jnp.float32),
            pltpu.VMEM_SHARED((NPAD,), jnp.float32),
        ],
    )
    def run(col_hbm, w_hbm, out_hbm, cidx, wv, zbuf, acc):
        c = lax.axis_index("c")
        s = lax.axis_index("s")
        wid = c * NS + s

        @pl.loop(0, SLICE_N // L)
        def _(i):
            zbuf[pl.ds(i * L, L)] = jnp.zeros((L,), jnp.float32)

        pltpu.sync_copy(zbuf, acc.at[pl.ds(s * SLICE_N, SLICE_N)])
        plsc.subcore_barrier()

        @pl.loop(0, CH_DEG // SB)
        def _(b):
            pltpu.sync_copy(col_hbm.at[wid, pl.ds(b * SB, SB)], cidx)
            pltpu.sync_copy(w_hbm.at[wid, pl.ds(b * SB, SB)], wv)
            for kk in range(SB):
                pltpu.sync_copy(wv.at[kk], acc.at[cidx.at[kk]], add=True)

        plsc.subcore_barrier()
        pltpu.sync_copy(
            acc.at[pl.ds(s * SLICE_N, SLICE_N)],
            out_hbm.at[pl.ds(c * NPAD + s * SLICE_N, SLICE_N)],
        )

    return run(col_d, w_d)


def _agg_sc(h_cat, row_t, col_t, w_t):
    @functools.partial(
        pl.kernel,
        out_type=jax.ShapeDtypeStruct((NC * NPAD, DH), jnp.float32),
        mesh=_sc_mesh(),
        compiler_params=_sc_params(),
        scratch_types=[
            pltpu.VMEM((SB, K), jnp.int32),
            pltpu.VMEM((SB, K), jnp.int32),
            pltpu.VMEM((SB, K), jnp.float32),
            pltpu.VMEM((K, DH), jnp.float32),
            pltpu.VMEM((ZR, DH), jnp.float32),
            pltpu.VMEM_SHARED((NPAD, DH), jnp.float32),
        ],
    )
    def run(h_hbm, r_hbm, c_hbm, w_hbm, out_hbm,
            ridx, cidx, wv, rows, zblk, acc):
        c = lax.axis_index("c")
        s = lax.axis_index("s")

        @pl.loop(0, ZR)
        def _(i):
            for j in range(DH // L):
                zblk[i, pl.ds(j * L, L)] = jnp.zeros((L,), jnp.float32)

        for i in range(SLICE_N // ZR):
            pltpu.sync_copy(zblk, acc.at[pl.ds(s * SLICE_N + i * ZR, ZR)])

        plsc.subcore_barrier()

        @pl.loop(0, NB_AGG)
        def _(b):
            pltpu.sync_copy(r_hbm.at[s, pl.ds(b * SB, SB)], ridx)
            pltpu.sync_copy(c_hbm.at[s, pl.ds(b * SB, SB)], cidx)
            pltpu.sync_copy(w_hbm.at[s, pl.ds(b * SB, SB)], wv)

            @pl.loop(0, SB)
            def _(kk):
                for j in range(K // L):
                    sl = (kk, pl.ds(j * L, L))
                    ridx[sl] = ridx[sl] + c * N

            @pl.loop(0, SB)
            def _(kk):
                pltpu.sync_copy(h_hbm.at[ridx.at[kk]], rows)

                @pl.loop(0, K)
                def _(e):
                    ws = plsc.load_gather(
                        wv,
                        [jnp.full((L,), kk, jnp.int32),
                         jnp.full((L,), e, jnp.int32)],
                    )
                    for j in range(DH // L):
                        sl = (e, pl.ds(j * L, L))
                        rows[sl] = rows[sl] * ws

                pltpu.sync_copy(rows, acc.at[cidx.at[kk]], add=True)

        plsc.subcore_barrier()
        pltpu.sync_copy(
            acc.at[pl.ds(s * SLICE_N, SLICE_N)],
            out_hbm.at[pl.ds(c * NPAD + s * SLICE_N, SLICE_N)],
        )

    return run(h_cat, row_t, col_t, w_t)


def _mm1_tc(x, w1):
    def body(x_ref, w_ref, o_ref):
        o_ref[...] = jnp.dot(x_ref[...], w_ref[...],
                             preferred_element_type=jnp.float32)

    return pl.pallas_call(
        body, out_shape=jax.ShapeDtypeStruct((N, DOUT), jnp.float32)
    )(x, w1)


def _prep1_tc(deg_a, deg_b, h1):
    def body(da, db, h, dinv_o, hp_o):
        dinv = lax.rsqrt(da[...] + db[...] + 1.0)
        dinv_o[...] = dinv
        hp_o[0] = dinv * h[:, :DH]
        hp_o[1] = dinv * h[:, DH:]

    return pl.pallas_call(
        body,
        out_shape=(
            jax.ShapeDtypeStruct((N, 1), jnp.float32),
            jax.ShapeDtypeStruct((2, N, DH), jnp.float32),
        ),
    )(deg_a, deg_b, h1)


def _layer2_tc(acc1, h1p, dinv, b1, w2):
    def body(a, hp, dv, b, w, o):
        d = dv[...]
        x0 = jnp.maximum(d * (a[0:N] + hp[0]) + b[0:1, :DH], 0.0)
        x1 = jnp.maximum(d * (a[NPAD:NPAD + N] + hp[1]) + b[0:1, DH:], 0.0)
        x2 = jnp.concatenate([x0, x1], axis=1)
        h2 = jnp.dot(x2, w[...], preferred_element_type=jnp.float32)
        o[0] = d * h2[:, :DH]
        o[1] = d * h2[:, DH:]

    return pl.pallas_call(
        body,
        out_shape=jax.ShapeDtypeStruct((2, N, DH), jnp.float32),
    )(acc1, h1p, dinv, b1, w2)


def _final_tc(acc2, h2p, dinv, b2, batch_row):
    def body(a, hp, dv, b, bt, o):
        d = dv[...]
        x0 = jnp.maximum(d * (a[0:N] + hp[0]) + b[0:1, :DH], 0.0)
        x1 = jnp.maximum(d * (a[NPAD:NPAD + N] + hp[1]) + b[0:1, DH:], 0.0)
        x3 = jnp.concatenate([x0, x1], axis=1)
        onehot = (lax.broadcasted_iota(jnp.int32, (G, N), 0)
                  == bt[...]).astype(jnp.float32)
        seg = jnp.dot(onehot, x3, preferred_element_type=jnp.float32)
        cnt = jnp.dot(onehot, jnp.ones((N, 1), jnp.float32),
                      preferred_element_type=jnp.float32)
        o[...] = seg / jnp.maximum(cnt, 1.0)

    return pl.pallas_call(
        body,
        out_shape=jax.ShapeDtypeStruct((G, DOUT), jnp.float32),
    )(acc2, h2p, dinv, b2, batch_row)


def kernel(node_features, edge_index, edge_weight, batch, W1, b1, W2, b2):
    row = edge_index[0].astype(jnp.int32)
    col = edge_index[1].astype(jnp.int32)
    w = edge_weight.astype(jnp.float32)

    pad = EPAD - E
    rowp = jnp.concatenate([row, jnp.zeros((pad,), jnp.int32)])
    colp = jnp.concatenate([col, jnp.zeros((pad,), jnp.int32)])
    wp = jnp.concatenate([w, jnp.zeros((pad,), jnp.float32)])

    row_t = rowp.reshape(NS, CH_AGG, K)
    col_t = colp.reshape(NS, CH_AGG, K)
    w_t = wp.reshape(NS, CH_AGG, K)
    col_d = colp.reshape(NC * NS, CH_DEG, K)
    w_d = wp.reshape(NC * NS, CH_DEG, K)

    h1 = _mm1_tc(node_features, W1)
    degflat = _deg_sc(col_d, w_d)
    deg_a = degflat[:N].reshape(N, 1)
    deg_b = degflat[NPAD:NPAD + N].reshape(N, 1)

    dinv, h1p = _prep1_tc(deg_a, deg_b, h1)
    acc1 = _agg_sc(h1p.reshape(2 * N, DH), row_t, col_t, w_t)

    h2p = _layer2_tc(acc1, h1p, dinv, b1.reshape(1, DOUT), W2)
    acc2 = _agg_sc(h2p.reshape(2 * N, DH), row_t, col_t, w_t)

    return _final_tc(acc2, h2p, dinv, b2.reshape(1, DOUT),
                     batch.astype(jnp.int32).reshape(1, N))


# speedup vs baseline: 7.4375x; 1.2887x over previous
import dataclasses
import functools

import jax
import jax.numpy as jnp
from jax import lax
from jax.experimental import pallas as pl
from jax.experimental.pallas import tpu as pltpu
from jax.experimental.pallas import tpu_sc as plsc

N = 10000
E = 320000
DIN = 128
DOUT = 256
DH = DOUT // 2
G = 128

NC = 2
NS = 16
L = 16
K = 128
SB = 16

EPAD = 327680
CH_DEG = EPAD // (NC * NS) // K
CH_AGG = EPAD // NS // K
NB_AGG = CH_AGG // SB

NPAD = 10240
SLICE_N = NPAD // NS
ZR = 64


def _sc_mesh():
    return plsc.VectorSubcoreMesh(core_axis_name="c", subcore_axis_name="s")


def _sc_params():
    cp = pltpu.CompilerParams()
    if "needs_layout_passes" in pltpu.CompilerParams.__dataclass_fields__:
        cp = dataclasses.replace(cp, needs_layout_passes=False)
    return cp


def _deg_sc(col_d, w_d):
    @functools.partial(
        pl.kernel,
        out_type=jax.ShapeDtypeStruct((NC * NPAD,), jnp.float32),
        mesh=_sc_mesh(),
        compiler_params=_sc_params(),
        scratch_types=[
            pltpu.VMEM((SB, K), jnp.int32),
            pltpu.VMEM((SB, K), jnp.float32),
            pltpu.VMEM((SLICE_N,), jnp.float32),
            pltpu.VMEM_SHARED((NPAD,), jnp.float32),
        ],
    )
    def run(col_hbm, w_hbm, out_hbm, cidx, wv, zbuf, acc):
        c = lax.axis_index("c")
        s = lax.axis_index("s")
        wid = c * NS + s

        @pl.loop(0, SLICE_N // L)
        def _(i):
            zbuf[pl.ds(i * L, L)] = jnp.zeros((L,), jnp.float32)

        pltpu.sync_copy(zbuf, acc.at[pl.ds(s * SLICE_N, SLICE_N)])
        plsc.subcore_barrier()

        @pl.loop(0, CH_DEG // SB)
        def _(b):
            pltpu.sync_copy(col_hbm.at[wid, pl.ds(b * SB, SB)], cidx)
            pltpu.sync_copy(w_hbm.at[wid, pl.ds(b * SB, SB)], wv)
            for kk in range(SB):
                pltpu.sync_copy(wv.at[kk], acc.at[cidx.at[kk]], add=True)

        plsc.subcore_barrier()
        pltpu.sync_copy(
            acc.at[pl.ds(s * SLICE_N, SLICE_N)],
            out_hbm.at[pl.ds(c * NPAD + s * SLICE_N, SLICE_N)],
        )

    return run(col_d, w_d)


def _agg_sc(h_cat, row_t, col_t, w_t):
    @functools.partial(
        pl.kernel,
        out_type=jax.ShapeDtypeStruct((NC * NPAD, DH), jnp.float32),
        mesh=_sc_mesh(),
        compiler_params=_sc_params(),
        scratch_types=[
            pltpu.VMEM((SB, K), jnp.int32),
            pltpu.VMEM((SB, K), jnp.int32),
            pltpu.VMEM((SB, K), jnp.float32),
            pltpu.VMEM((K, DH), jnp.float32),
            pltpu.VMEM((K, DH), jnp.float32),
            pltpu.VMEM((ZR, DH), jnp.float32),
            pltpu.VMEM_SHARED((NPAD, DH), jnp.float32),
            pltpu.SemaphoreType.DMA,
            pltpu.SemaphoreType.DMA,
            pltpu.SemaphoreType.DMA,
            pltpu.SemaphoreType.DMA,
        ],
    )
    def run(h_hbm, r_hbm, c_hbm, w_hbm, out_hbm,
            ridx, cidx, wv, rows0, rows1, zblk, acc,
            gsem0, gsem1, ssem0, ssem1):
        c = lax.axis_index("c")
        s = lax.axis_index("s")

        @pl.loop(0, ZR)
        def _(i):
            for j in range(DH // L):
                zblk[i, pl.ds(j * L, L)] = jnp.zeros((L,), jnp.float32)

        for i in range(SLICE_N // ZR):
            pltpu.sync_copy(zblk, acc.at[pl.ds(s * SLICE_N + i * ZR, ZR)])

        plsc.subcore_barrier()
        bufs = ((rows0, gsem0, ssem0), (rows1, gsem1, ssem1))

        @pl.loop(0, NB_AGG)
        def _(b):
            pltpu.sync_copy(r_hbm.at[s, pl.ds(b * SB, SB)], ridx)
            pltpu.sync_copy(c_hbm.at[s, pl.ds(b * SB, SB)], cidx)
            pltpu.sync_copy(w_hbm.at[s, pl.ds(b * SB, SB)], wv)

            @pl.loop(0, SB)
            def _(kk):
                for j in range(K // L):
                    sl = (kk, pl.ds(j * L, L))
                    ridx[sl] = ridx[sl] + c * N

            gathers = {}
            scatters = {}
            buf0 = bufs[0][0]
            gathers[0] = pltpu.async_copy(h_hbm.at[ridx.at[0]], buf0, bufs[0][1])
            for kk in range(SB):
                buf, gsem, ssem = bufs[kk % 2]
                obuf, ogsem, _ = bufs[(kk + 1) % 2]
                gathers[kk].wait()
                if kk + 1 < SB:
                    if kk >= 1:
                        scatters[kk - 1].wait()
                    gathers[kk + 1] = pltpu.async_copy(
                        h_hbm.at[ridx.at[kk + 1]], obuf, ogsem)

                @pl.loop(0, K)
                def _(e):
                    ws = plsc.load_gather(
                        wv,
                        [jnp.full((L,), kk, jnp.int32),
                         jnp.full((L,), e, jnp.int32)],
                    )
                    for j in range(DH // L):
                        sl = (e, pl.ds(j * L, L))
                        buf[sl] = buf[sl] * ws

                scatters[kk] = pltpu.async_copy(
                    buf, acc.at[cidx.at[kk]], ssem, add=True)
            scatters[SB - 2].wait()
            scatters[SB - 1].wait()

        plsc.subcore_barrier()
        pltpu.sync_copy(
            acc.at[pl.ds(s * SLICE_N, SLICE_N)],
            out_hbm.at[pl.ds(c * NPAD + s * SLICE_N, SLICE_N)],
        )

    return run(h_cat, row_t, col_t, w_t)


def _mm1_tc(x, w1):
    def body(x_ref, w_ref, o_ref):
        o_ref[...] = jnp.dot(x_ref[...], w_ref[...],
                             preferred_element_type=jnp.float32)

    return pl.pallas_call(
        body, out_shape=jax.ShapeDtypeStruct((N, DOUT), jnp.float32)
    )(x, w1)


def _prep1_tc(deg_a, deg_b, h1):
    def body(da, db, h, dinv_o, hp_o):
        dinv = lax.rsqrt(da[...] + db[...] + 1.0)
        dinv_o[...] = dinv
        hp_o[0] = dinv * h[:, :DH]
        hp_o[1] = dinv * h[:, DH:]

    return pl.pallas_call(
        body,
        out_shape=(
            jax.ShapeDtypeStruct((N, 1), jnp.float32),
            jax.ShapeDtypeStruct((2, N, DH), jnp.float32),
        ),
    )(deg_a, deg_b, h1)


def _layer2_tc(acc1, h1p, dinv, b1, w2):
    def body(a, hp, dv, b, w, o):
        d = dv[...]
        x0 = jnp.maximum(d * (a[0:N] + hp[0]) + b[0:1, :DH], 0.0)
        x1 = jnp.maximum(d * (a[NPAD:NPAD + N] + hp[1]) + b[0:1, DH:], 0.0)
        x2 = jnp.concatenate([x0, x1], axis=1)
        h2 = jnp.dot(x2, w[...], preferred_element_type=jnp.float32)
        o[0] = d * h2[:, :DH]
        o[1] = d * h2[:, DH:]

    return pl.pallas_call(
        body,
        out_shape=jax.ShapeDtypeStruct((2, N, DH), jnp.float32),
    )(acc1, h1p, dinv, b1, w2)


def _final_tc(acc2, h2p, dinv, b2, batch_row):
    def body(a, hp, dv, b, bt, o):
        d = dv[...]
        x0 = jnp.maximum(d * (a[0:N] + hp[0]) + b[0:1, :DH], 0.0)
        x1 = jnp.maximum(d * (a[NPAD:NPAD + N] + hp[1]) + b[0:1, DH:], 0.0)
        x3 = jnp.concatenate([x0, x1], axis=1)
        onehot = (lax.broadcasted_iota(jnp.int32, (G, N), 0)
                  == bt[...]).astype(jnp.float32)
        seg = jnp.dot(onehot, x3, preferred_element_type=jnp.float32)
        cnt = jnp.dot(onehot, jnp.ones((N, 1), jnp.float32),
                      preferred_element_type=jnp.float32)
        o[...] = seg / jnp.maximum(cnt, 1.0)

    return pl.pallas_call(
        body,
        out_shape=jax.ShapeDtypeStruct((G, DOUT), jnp.float32),
    )(acc2, h2p, dinv, b2, batch_row)


def kernel(node_features, edge_index, edge_weight, batch, W1, b1, W2, b2):
    row = edge_index[0].astype(jnp.int32)
    col = edge_index[1].astype(jnp.int32)
    w = edge_weight.astype(jnp.float32)

    pad = EPAD - E
    rowp = jnp.concatenate([row, jnp.zeros((pad,), jnp.int32)])
    colp = jnp.concatenate([col, jnp.zeros((pad,), jnp.int32)])
    wp = jnp.concatenate([w, jnp.zeros((pad,), jnp.float32)])

    row_t = rowp.reshape(NS, CH_AGG, K)
    col_t = colp.reshape(NS, CH_AGG, K)
    w_t = wp.reshape(NS, CH_AGG, K)
    col_d = colp.reshape(NC * NS, CH_DEG, K)
    w_d = wp.reshape(NC * NS, CH_DEG, K)

    h1 = _mm1_tc(node_features, W1)
    degflat = _deg_sc(col_d, w_d)
    deg_a = degflat[:N].reshape(N, 1)
    deg_b = degflat[NPAD:NPAD + N].reshape(N, 1)

    dinv, h1p = _prep1_tc(deg_a, deg_b, h1)
    acc1 = _agg_sc(h1p.reshape(2 * N, DH), row_t, col_t, w_t)

    h2p = _layer2_tc(acc1, h1p, dinv, b1.reshape(1, DOUT), W2)
    acc2 = _agg_sc(h2p.reshape(2 * N, DH), row_t, col_t, w_t)

    return _final_tc(acc2, h2p, dinv, b2.reshape(1, DOUT),
                     batch.astype(jnp.int32).reshape(1, N))


# speedup vs baseline: 8.5771x; 1.1532x over previous
import dataclasses
import functools

import jax
import jax.numpy as jnp
from jax import lax
from jax.experimental import pallas as pl
from jax.experimental.pallas import tpu as pltpu
from jax.experimental.pallas import tpu_sc as plsc

N = 10000
E = 320000
DIN = 128
DOUT = 256
DH = DOUT // 2
G = 128

NC = 2
NS = 16
L = 16
K = 128
SB = 16
KA = 64

EPAD = 327680
CH_DEG = EPAD // (NC * NS) // K
CH_AGG = EPAD // NS // KA
NB_AGG = CH_AGG // SB

NPAD = 10240
SLICE_N = NPAD // NS
ZR = 64


def _sc_mesh():
    return plsc.VectorSubcoreMesh(core_axis_name="c", subcore_axis_name="s")


def _sc_params():
    cp = pltpu.CompilerParams()
    if "needs_layout_passes" in pltpu.CompilerParams.__dataclass_fields__:
        cp = dataclasses.replace(cp, needs_layout_passes=False)
    return cp


def _deg_sc(col_d, w_d):
    @functools.partial(
        pl.kernel,
        out_type=jax.ShapeDtypeStruct((NC * NPAD,), jnp.float32),
        mesh=_sc_mesh(),
        compiler_params=_sc_params(),
        scratch_types=[
            pltpu.VMEM((SB, K), jnp.int32),
            pltpu.VMEM((SB, K), jnp.float32),
            pltpu.VMEM((SLICE_N,), jnp.float32),
            pltpu.VMEM_SHARED((NPAD,), jnp.float32),
        ],
    )
    def run(col_hbm, w_hbm, out_hbm, cidx, wv, zbuf, acc):
        c = lax.axis_index("c")
        s = lax.axis_index("s")
        wid = c * NS + s

        @pl.loop(0, SLICE_N // L)
        def _(i):
            zbuf[pl.ds(i * L, L)] = jnp.zeros((L,), jnp.float32)

        pltpu.sync_copy(zbuf, acc.at[pl.ds(s * SLICE_N, SLICE_N)])
        plsc.subcore_barrier()

        @pl.loop(0, CH_DEG // SB)
        def _(b):
            pltpu.sync_copy(col_hbm.at[wid, pl.ds(b * SB, SB)], cidx)
            pltpu.sync_copy(w_hbm.at[wid, pl.ds(b * SB, SB)], wv)
            for kk in range(SB):
                pltpu.sync_copy(wv.at[kk], acc.at[cidx.at[kk]], add=True)

        plsc.subcore_barrier()
        pltpu.sync_copy(
            acc.at[pl.ds(s * SLICE_N, SLICE_N)],
            out_hbm.at[pl.ds(c * NPAD + s * SLICE_N, SLICE_N)],
        )

    return run(col_d, w_d)


def _agg_sc(h_cat, row_t, col_t, w_t):
    @functools.partial(
        pl.kernel,
        out_type=jax.ShapeDtypeStruct((NC * NPAD, DH), jnp.float32),
        mesh=_sc_mesh(),
        compiler_params=_sc_params(),
        scratch_types=[
            pltpu.VMEM((SB, KA), jnp.int32),
            pltpu.VMEM((SB, KA), jnp.int32),
            pltpu.VMEM((SB, KA), jnp.float32),
            pltpu.VMEM((KA, DH), jnp.float32),
            pltpu.VMEM((KA, DH), jnp.float32),
            pltpu.VMEM((KA, DH), jnp.float32),
            pltpu.VMEM((ZR, DH), jnp.float32),
            pltpu.VMEM_SHARED((NPAD, DH), jnp.float32),
            pltpu.SemaphoreType.DMA,
            pltpu.SemaphoreType.DMA,
            pltpu.SemaphoreType.DMA,
            pltpu.SemaphoreType.DMA,
            pltpu.SemaphoreType.DMA,
            pltpu.SemaphoreType.DMA,
        ],
    )
    def run(h_hbm, r_hbm, c_hbm, w_hbm, out_hbm,
            ridx, cidx, wv, rows0, rows1, rows2, zblk, acc,
            gsem0, gsem1, gsem2, ssem0, ssem1, ssem2):
        c = lax.axis_index("c")
        s = lax.axis_index("s")

        @pl.loop(0, ZR)
        def _(i):
            for j in range(DH // L):
                zblk[i, pl.ds(j * L, L)] = jnp.zeros((L,), jnp.float32)

        for i in range(SLICE_N // ZR):
            pltpu.sync_copy(zblk, acc.at[pl.ds(s * SLICE_N + i * ZR, ZR)])

        plsc.subcore_barrier()
        bufs = ((rows0, gsem0, ssem0), (rows1, gsem1, ssem1),
                (rows2, gsem2, ssem2))

        @pl.loop(0, NB_AGG)
        def _(b):
            pltpu.sync_copy(r_hbm.at[s, pl.ds(b * SB, SB)], ridx)
            pltpu.sync_copy(c_hbm.at[s, pl.ds(b * SB, SB)], cidx)
            pltpu.sync_copy(w_hbm.at[s, pl.ds(b * SB, SB)], wv)

            @pl.loop(0, SB)
            def _(kk):
                for j in range(KA // L):
                    sl = (kk, pl.ds(j * L, L))
                    ridx[sl] = ridx[sl] + c * N

            gathers = {}
            scatters = {}
            gathers[0] = pltpu.async_copy(
                h_hbm.at[ridx.at[0]], bufs[0][0], bufs[0][1])
            gathers[1] = pltpu.async_copy(
                h_hbm.at[ridx.at[1]], bufs[1][0], bufs[1][1])
            for kk in range(SB):
                buf, gsem, ssem = bufs[kk % 3]
                gathers[kk].wait()

                @pl.loop(0, KA)
                def _(e):
                    ws = plsc.load_gather(
                        wv,
                        [jnp.full((L,), kk, jnp.int32),
                         jnp.full((L,), e, jnp.int32)],
                    )
                    for j in range(DH // L):
                        sl = (e, pl.ds(j * L, L))
                        buf[sl] = buf[sl] * ws

                scatters[kk] = pltpu.async_copy(
                    buf, acc.at[cidx.at[kk]], ssem, add=True)
                if kk + 2 < SB:
                    nbuf, ngsem, _ = bufs[(kk + 2) % 3]
                    if kk >= 1:
                        scatters[kk - 1].wait()
                    gathers[kk + 2] = pltpu.async_copy(
                        h_hbm.at[ridx.at[kk + 2]], nbuf, ngsem)
            scatters[SB - 3].wait()
            scatters[SB - 2].wait()
            scatters[SB - 1].wait()

        plsc.subcore_barrier()
        pltpu.sync_copy(
            acc.at[pl.ds(s * SLICE_N, SLICE_N)],
            out_hbm.at[pl.ds(c * NPAD + s * SLICE_N, SLICE_N)],
        )

    return run(h_cat, row_t, col_t, w_t)


def _mm1_tc(x, w1):
    def body(x_ref, w_ref, o_ref):
        o_ref[...] = jnp.dot(x_ref[...], w_ref[...],
                             preferred_element_type=jnp.float32)

    return pl.pallas_call(
        body, out_shape=jax.ShapeDtypeStruct((N, DOUT), jnp.float32)
    )(x, w1)


def _prep1_tc(deg_a, deg_b, h1):
    def body(da, db, h, dinv_o, hp_o):
        dinv = lax.rsqrt(da[...] + db[...] + 1.0)
        dinv_o[...] = dinv
        hp_o[0] = dinv * h[:, :DH]
        hp_o[1] = dinv * h[:, DH:]

    return pl.pallas_call(
        body,
        out_shape=(
            jax.ShapeDtypeStruct((N, 1), jnp.float32),
            jax.ShapeDtypeStruct((2, N, DH), jnp.float32),
        ),
    )(deg_a, deg_b, h1)


def _layer2_tc(acc1, h1p, dinv, b1, w2):
    def body(a, hp, dv, b, w, o):
        d = dv[...]
        x0 = jnp.maximum(d * (a[0:N] + hp[0]) + b[0:1, :DH], 0.0)
        x1 = jnp.maximum(d * (a[NPAD:NPAD + N] + hp[1]) + b[0:1, DH:], 0.0)
        x2 = jnp.concatenate([x0, x1], axis=1)
        h2 = jnp.dot(x2, w[...], preferred_element_type=jnp.float32)
        o[0] = d * h2[:, :DH]
        o[1] = d * h2[:, DH:]

    return pl.pallas_call(
        body,
        out_shape=jax.ShapeDtypeStruct((2, N, DH), jnp.float32),
    )(acc1, h1p, dinv, b1, w2)


def _final_tc(acc2, h2p, dinv, b2, batch_row):
    def body(a, hp, dv, b, bt, o):
        d = dv[...]
        x0 = jnp.maximum(d * (a[0:N] + hp[0]) + b[0:1, :DH], 0.0)
        x1 = jnp.maximum(d * (a[NPAD:NPAD + N] + hp[1]) + b[0:1, DH:], 0.0)
        x3 = jnp.concatenate([x0, x1], axis=1)
        onehot = (lax.broadcasted_iota(jnp.int32, (G, N), 0)
                  == bt[...]).astype(jnp.float32)
        seg = jnp.dot(onehot, x3, preferred_element_type=jnp.float32)
        cnt = jnp.dot(onehot, jnp.ones((N, 1), jnp.float32),
                      preferred_element_type=jnp.float32)
        o[...] = seg / jnp.maximum(cnt, 1.0)

    return pl.pallas_call(
        body,
        out_shape=jax.ShapeDtypeStruct((G, DOUT), jnp.float32),
    )(acc2, h2p, dinv, b2, batch_row)


def kernel(node_features, edge_index, edge_weight, batch, W1, b1, W2, b2):
    row = edge_index[0].astype(jnp.int32)
    col = edge_index[1].astype(jnp.int32)
    w = edge_weight.astype(jnp.float32)

    pad = EPAD - E
    rowp = jnp.concatenate([row, jnp.zeros((pad,), jnp.int32)])
    colp = jnp.concatenate([col, jnp.zeros((pad,), jnp.int32)])
    wp = jnp.concatenate([w, jnp.zeros((pad,), jnp.float32)])

    row_t = rowp.reshape(NS, CH_AGG, KA)
    col_t = colp.reshape(NS, CH_AGG, KA)
    w_t = wp.reshape(NS, CH_AGG, KA)
    col_d = colp.reshape(NC * NS, CH_DEG, K)
    w_d = wp.reshape(NC * NS, CH_DEG, K)

    h1 = _mm1_tc(node_features, W1)
    degflat = _deg_sc(col_d, w_d)
    deg_a = degflat[:N].reshape(N, 1)
    deg_b = degflat[NPAD:NPAD + N].reshape(N, 1)

    dinv, h1p = _prep1_tc(deg_a, deg_b, h1)
    acc1 = _agg_sc(h1p.reshape(2 * N, DH), row_t, col_t, w_t)

    h2p = _layer2_tc(acc1, h1p, dinv, b1.reshape(1, DOUT), W2)
    acc2 = _agg_sc(h2p.reshape(2 * N, DH), row_t, col_t, w_t)

    return _final_tc(acc2, h2p, dinv, b2.reshape(1, DOUT),
                     batch.astype(jnp.int32).reshape(1, N))


# speedup vs baseline: 9.0370x; 1.0536x over previous
import dataclasses
import functools

import jax
import jax.numpy as jnp
from jax import lax
from jax.experimental import pallas as pl
from jax.experimental.pallas import tpu as pltpu
from jax.experimental.pallas import tpu_sc as plsc

N = 10000
E = 320000
DIN = 128
DOUT = 256
DH = DOUT // 2
G = 128

NC = 2
NS = 16
L = 16
K = 128
SB = 40
KA = 64

EPAD = 327680
CH_DEG = EPAD // (NC * NS) // K
CH_AGG = EPAD // NS // KA
NB_AGG = CH_AGG // SB

NPAD = 10240
SLICE_N = NPAD // NS
ZR = 64


def _sc_mesh():
    return plsc.VectorSubcoreMesh(core_axis_name="c", subcore_axis_name="s")


def _sc_params():
    cp = pltpu.CompilerParams()
    if "needs_layout_passes" in pltpu.CompilerParams.__dataclass_fields__:
        cp = dataclasses.replace(cp, needs_layout_passes=False)
    return cp


def _deg_sc(col_d, w_d):
    @functools.partial(
        pl.kernel,
        out_type=jax.ShapeDtypeStruct((NC * NPAD,), jnp.float32),
        mesh=_sc_mesh(),
        compiler_params=_sc_params(),
        scratch_types=[
            pltpu.VMEM((SB, K), jnp.int32),
            pltpu.VMEM((SB, K), jnp.float32),
            pltpu.VMEM((SLICE_N,), jnp.float32),
            pltpu.VMEM_SHARED((NPAD,), jnp.float32),
        ],
    )
    def run(col_hbm, w_hbm, out_hbm, cidx, wv, zbuf, acc):
        c = lax.axis_index("c")
        s = lax.axis_index("s")
        wid = c * NS + s

        @pl.loop(0, SLICE_N // L)
        def _(i):
            zbuf[pl.ds(i * L, L)] = jnp.zeros((L,), jnp.float32)

        pltpu.sync_copy(zbuf, acc.at[pl.ds(s * SLICE_N, SLICE_N)])
        plsc.subcore_barrier()

        @pl.loop(0, CH_DEG // SB)
        def _(b):
            pltpu.sync_copy(col_hbm.at[wid, pl.ds(b * SB, SB)], cidx)
            pltpu.sync_copy(w_hbm.at[wid, pl.ds(b * SB, SB)], wv)
            for kk in range(SB):
                pltpu.sync_copy(wv.at[kk], acc.at[cidx.at[kk]], add=True)

        plsc.subcore_barrier()
        pltpu.sync_copy(
            acc.at[pl.ds(s * SLICE_N, SLICE_N)],
            out_hbm.at[pl.ds(c * NPAD + s * SLICE_N, SLICE_N)],
        )

    return run(col_d, w_d)


def _agg_sc(h_cat, row_t, col_t, w_t):
    @functools.partial(
        pl.kernel,
        out_type=jax.ShapeDtypeStruct((NC * NPAD, DH), jnp.float32),
        mesh=_sc_mesh(),
        compiler_params=_sc_params(),
        scratch_types=[
            pltpu.VMEM((SB, KA), jnp.int32),
            pltpu.VMEM((SB, KA), jnp.int32),
            pltpu.VMEM((SB, KA), jnp.float32),
            pltpu.VMEM((KA, DH), jnp.float32),
            pltpu.VMEM((KA, DH), jnp.float32),
            pltpu.VMEM((KA, DH), jnp.float32),
            pltpu.VMEM((ZR, DH), jnp.float32),
            pltpu.VMEM_SHARED((NPAD, DH), jnp.float32),
            pltpu.SemaphoreType.DMA,
            pltpu.SemaphoreType.DMA,
            pltpu.SemaphoreType.DMA,
            pltpu.SemaphoreType.DMA,
            pltpu.SemaphoreType.DMA,
            pltpu.SemaphoreType.DMA,
        ],
    )
    def run(h_hbm, r_hbm, c_hbm, w_hbm, out_hbm,
            ridx, cidx, wv, rows0, rows1, rows2, zblk, acc,
            gsem0, gsem1, gsem2, ssem0, ssem1, ssem2):
        c = lax.axis_index("c")
        s = lax.axis_index("s")

        @pl.loop(0, ZR)
        def _(i):
            for j in range(DH // L):
                zblk[i, pl.ds(j * L, L)] = jnp.zeros((L,), jnp.float32)

        for i in range(SLICE_N // ZR):
            pltpu.sync_copy(zblk, acc.at[pl.ds(s * SLICE_N + i * ZR, ZR)])

        plsc.subcore_barrier()
        bufs = ((rows0, gsem0, ssem0), (rows1, gsem1, ssem1),
                (rows2, gsem2, ssem2))

        @pl.loop(0, NB_AGG)
        def _(b):
            pltpu.sync_copy(r_hbm.at[s, pl.ds(b * SB, SB)], ridx)
            pltpu.sync_copy(c_hbm.at[s, pl.ds(b * SB, SB)], cidx)
            pltpu.sync_copy(w_hbm.at[s, pl.ds(b * SB, SB)], wv)

            @pl.loop(0, SB)
            def _(kk):
                for j in range(KA // L):
                    sl = (kk, pl.ds(j * L, L))
                    ridx[sl] = ridx[sl] + c * N

            gathers = {}
            scatters = {}
            gathers[0] = pltpu.async_copy(
                h_hbm.at[ridx.at[0]], bufs[0][0], bufs[0][1])
            gathers[1] = pltpu.async_copy(
                h_hbm.at[ridx.at[1]], bufs[1][0], bufs[1][1])
            for kk in range(SB):
                buf, gsem, ssem = bufs[kk % 3]
                gathers[kk].wait()

                @pl.loop(0, KA)
                def _(e):
                    ws = plsc.load_gather(
                        wv,
                        [jnp.full((L,), kk, jnp.int32),
                         jnp.full((L,), e, jnp.int32)],
                    )
                    for j in range(DH // L):
                        sl = (e, pl.ds(j * L, L))
                        buf[sl] = buf[sl] * ws

                scatters[kk] = pltpu.async_copy(
                    buf, acc.at[cidx.at[kk]], ssem, add=True)
                if kk + 2 < SB:
                    nbuf, ngsem, _ = bufs[(kk + 2) % 3]
                    if kk >= 1:
                        scatters[kk - 1].wait()
                    gathers[kk + 2] = pltpu.async_copy(
                        h_hbm.at[ridx.at[kk + 2]], nbuf, ngsem)
            scatters[SB - 3].wait()
            scatters[SB - 2].wait()
            scatters[SB - 1].wait()

        plsc.subcore_barrier()
        pltpu.sync_copy(
            acc.at[pl.ds(s * SLICE_N, SLICE_N)],
            out_hbm.at[pl.ds(c * NPAD + s * SLICE_N, SLICE_N)],
        )

    return run(h_cat, row_t, col_t, w_t)


def _mm1_tc(x, w1):
    def body(x_ref, w_ref, o_ref):
        o_ref[...] = jnp.dot(x_ref[...], w_ref[...],
                             preferred_element_type=jnp.float32)

    return pl.pallas_call(
        body, out_shape=jax.ShapeDtypeStruct((N, DOUT), jnp.float32)
    )(x, w1)


def _prep1_tc(deg_a, deg_b, h1):
    def body(da, db, h, dinv_o, hp_o):
        dinv = lax.rsqrt(da[...] + db[...] + 1.0)
        dinv_o[...] = dinv
        hp_o[0] = dinv * h[:, :DH]
        hp_o[1] = dinv * h[:, DH:]

    return pl.pallas_call(
        body,
        out_shape=(
            jax.ShapeDtypeStruct((N, 1), jnp.float32),
            jax.ShapeDtypeStruct((2, N, DH), jnp.float32),
        ),
    )(deg_a, deg_b, h1)


def _layer2_tc(acc1, h1p, dinv, b1, w2):
    def body(a, hp, dv, b, w, o):
        d = dv[...]
        x0 = jnp.maximum(d * (a[0:N] + hp[0]) + b[0:1, :DH], 0.0)
        x1 = jnp.maximum(d * (a[NPAD:NPAD + N] + hp[1]) + b[0:1, DH:], 0.0)
        x2 = jnp.concatenate([x0, x1], axis=1)
        h2 = jnp.dot(x2, w[...], preferred_element_type=jnp.float32)
        o[0] = d * h2[:, :DH]
        o[1] = d * h2[:, DH:]

    return pl.pallas_call(
        body,
        out_shape=jax.ShapeDtypeStruct((2, N, DH), jnp.float32),
    )(acc1, h1p, dinv, b1, w2)


def _final_tc(acc2, h2p, dinv, b2, batch_row):
    def body(a, hp, dv, b, bt, o):
        d = dv[...]
        x0 = jnp.maximum(d * (a[0:N] + hp[0]) + b[0:1, :DH], 0.0)
        x1 = jnp.maximum(d * (a[NPAD:NPAD + N] + hp[1]) + b[0:1, DH:], 0.0)
        x3 = jnp.concatenate([x0, x1], axis=1)
        onehot = (lax.broadcasted_iota(jnp.int32, (G, N), 0)
                  == bt[...]).astype(jnp.float32)
        seg = jnp.dot(onehot, x3, preferred_element_type=jnp.float32)
        cnt = jnp.dot(onehot, jnp.ones((N, 1), jnp.float32),
                      preferred_element_type=jnp.float32)
        o[...] = seg / jnp.maximum(cnt, 1.0)

    return pl.pallas_call(
        body,
        out_shape=jax.ShapeDtypeStruct((G, DOUT), jnp.float32),
    )(acc2, h2p, dinv, b2, batch_row)


def kernel(node_features, edge_index, edge_weight, batch, W1, b1, W2, b2):
    row = edge_index[0].astype(jnp.int32)
    col = edge_index[1].astype(jnp.int32)
    w = edge_weight.astype(jnp.float32)

    pad = EPAD - E
    rowp = jnp.concatenate([row, jnp.zeros((pad,), jnp.int32)])
    colp = jnp.concatenate([col, jnp.zeros((pad,), jnp.int32)])
    wp = jnp.concatenate([w, jnp.zeros((pad,), jnp.float32)])

    row_t = rowp.reshape(NS, CH_AGG, KA)
    col_t = colp.reshape(NS, CH_AGG, KA)
    w_t = wp.reshape(NS, CH_AGG, KA)
    col_d = colp.reshape(NC * NS, CH_DEG, K)
    w_d = wp.reshape(NC * NS, CH_DEG, K)

    h1 = _mm1_tc(node_features, W1)
    degflat = _deg_sc(col_d, w_d)
    deg_a = degflat[:N].reshape(N, 1)
    deg_b = degflat[NPAD:NPAD + N].reshape(N, 1)

    dinv, h1p = _prep1_tc(deg_a, deg_b, h1)
    acc1 = _agg_sc(h1p.reshape(2 * N, DH), row_t, col_t, w_t)

    h2p = _layer2_tc(acc1, h1p, dinv, b1.reshape(1, DOUT), W2)
    acc2 = _agg_sc(h2p.reshape(2 * N, DH), row_t, col_t, w_t)

    return _final_tc(acc2, h2p, dinv, b2.reshape(1, DOUT),
                     batch.astype(jnp.int32).reshape(1, N))
